# baseline (speedup 1.0000x reference)
import jax
import jax.numpy as jnp
from jax.experimental import pallas as pl
from jax.experimental.pallas import tpu as pltpu

RING = 4
LOOK = 3


def _moe_kernel(x_ref, wg_ref, wn_ref, eps_ref, b1_ref, b2_ref,
                w1_hbm, w2_hbm, out_ref,
                hpart_ref, w_ref, buf1_ref, buf2_ref, sem1, sem2):
    n_chunks = w1_hbm.shape[0]
    d_half = w1_hbm.shape[1]
    d_hid = w1_hbm.shape[2]
    h_half = w2_hbm.shape[1]
    n_exp = wg_ref.shape[1]

    def cp1(i, slot):
        return pltpu.make_async_copy(w1_hbm.at[pl.ds(i, 1)],
                                     buf1_ref.at[pl.ds(slot, 1)],
                                     sem1.at[slot])

    def cp2(i, slot):
        return pltpu.make_async_copy(w2_hbm.at[pl.ds(i, 1)],
                                     buf2_ref.at[pl.ds(slot, 1)],
                                     sem2.at[slot])

    for k in range(LOOK):
        cp1(k, k).start()
        cp2(k, k).start()

    xv = x_ref[...]
    g = jnp.dot(xv, wg_ref[...], preferred_element_type=jnp.float32)
    n = jnp.dot(xv, wn_ref[...], preferred_element_type=jnp.float32)
    logits = g + jax.nn.softplus(n) * eps_ref[...]
    lane = jax.lax.broadcasted_iota(jnp.int32, logits.shape, 1)
    i1 = jnp.argmax(logits, axis=1)[:, None]
    v1 = jnp.max(logits, axis=1)[:, None]
    oh1 = lane == i1
    masked = jnp.where(oh1, -jnp.inf, logits)
    i2 = jnp.argmax(masked, axis=1)[:, None]
    v2 = jnp.max(masked, axis=1)[:, None]
    oh2 = lane == i2
    e2 = jnp.exp(v2 - v1)
    denom = 1.0 + e2
    w_ref[...] = jnp.where(oh1, 1.0 / denom, jnp.where(oh2, e2 / denom, 0.0))
    out_ref[...] = jnp.zeros_like(out_ref)

    xa = xv[:, :d_half]
    xb = xv[:, d_half:]

    for i in range(n_chunks):
        e = i // 2
        s = i % RING
        cp1(i, s).wait()
        cp2(i, s).wait()
        if i % 2 == 0:
            hpart_ref[...] = jnp.dot(xa, buf1_ref[s],
                                     preferred_element_type=jnp.float32)
        else:
            sp = (i - 1) % RING
            h = (hpart_ref[...]
                 + jnp.dot(xb, buf1_ref[s],
                           preferred_element_type=jnp.float32)
                 + b1_ref[e])
            rh = jnp.maximum(h, 0.0)
            acc = (jnp.dot(rh[:, :h_half], buf2_ref[sp],
                           preferred_element_type=jnp.float32)
                   + jnp.dot(rh[:, h_half:], buf2_ref[s],
                             preferred_element_type=jnp.float32))
            out_ref[...] += w_ref[:, e:e + 1] * (acc + b2_ref[e])
        if i + LOOK < n_chunks:
            ns = (i + LOOK) % RING
            cp1(i + LOOK, ns).start()
            cp2(i + LOOK, ns).start()


def kernel(x, Wg, Wnoise, W1, b1, W2, b2):
    b, c, d = x.shape
    n_exp, _, d_hid = W1.shape
    t = b * c
    x2 = x.reshape(t, d)
    eps = jax.random.normal(jax.random.key(42), (b, c, n_exp),
                            dtype=x.dtype).reshape(t, n_exp)
    nc = 2 * n_exp
    w1r = W1.reshape(nc, d // 2, d_hid)
    w2r = W2.reshape(nc, d_hid // 2, d)
    out = pl.pallas_call(
        _moe_kernel,
        in_specs=[
            pl.BlockSpec((t, d), lambda: (0, 0)),
            pl.BlockSpec((d, n_exp), lambda: (0, 0)),
            pl.BlockSpec((d, n_exp), lambda: (0, 0)),
            pl.BlockSpec((t, n_exp), lambda: (0, 0)),
            pl.BlockSpec((n_exp, 1, d_hid), lambda: (0, 0, 0)),
            pl.BlockSpec((n_exp, 1, d), lambda: (0, 0, 0)),
            pl.BlockSpec(memory_space=pltpu.MemorySpace.HBM),
            pl.BlockSpec(memory_space=pltpu.MemorySpace.HBM),
        ],
        out_specs=pl.BlockSpec((t, d), lambda: (0, 0)),
        out_shape=jax.ShapeDtypeStruct((t, d), x.dtype),
        scratch_shapes=[
            pltpu.VMEM((t, d_hid), jnp.float32),
            pltpu.VMEM((t, n_exp), jnp.float32),
            pltpu.VMEM((RING, d // 2, d_hid), jnp.float32),
            pltpu.VMEM((RING, d_hid // 2, d), jnp.float32),
            pltpu.SemaphoreType.DMA((RING,)),
            pltpu.SemaphoreType.DMA((RING,)),
        ],
    )(x2, Wg.T, Wnoise.T, eps, b1[:, None, :], b2[:, None, :], w1r, w2r)
    return out.reshape(b, c, d)


# speedup vs baseline: 1.0169x; 1.0169x over previous
import jax
import jax.numpy as jnp
from jax.experimental import pallas as pl
from jax.experimental.pallas import tpu as pltpu

RING = 8
LOOK1 = 6
LAG = 4


def _moe_kernel(x_ref, wg_ref, wn_ref, eps_ref, b1_ref, b2_ref,
                w1_hbm, w2_hbm, out_ref, w_ref, buf1_ref, buf2_ref,
                sem1, sem2):
    n_chunks = w1_hbm.shape[0]
    d_q = w1_hbm.shape[1]
    h_q = w2_hbm.shape[1]

    def cp1(i, slot):
        return pltpu.make_async_copy(w1_hbm.at[pl.ds(i, 1)],
                                     buf1_ref.at[pl.ds(slot, 1)],
                                     sem1.at[slot])

    def cp2(i, slot):
        return pltpu.make_async_copy(w2_hbm.at[pl.ds(i, 1)],
                                     buf2_ref.at[pl.ds(slot, 1)],
                                     sem2.at[slot])

    for k in range(LOOK1):
        cp1(k, k).start()
    for k in range(2):
        cp2(k, k).start()

    xv = x_ref[...]
    g = jnp.dot(xv, wg_ref[...], preferred_element_type=jnp.float32)
    n = jnp.dot(xv, wn_ref[...], preferred_element_type=jnp.float32)
    logits = g + jax.nn.softplus(n) * eps_ref[...]
    lane = jax.lax.broadcasted_iota(jnp.int32, logits.shape, 1)
    i1 = jnp.argmax(logits, axis=1)[:, None]
    v1 = jnp.max(logits, axis=1)[:, None]
    oh1 = lane == i1
    masked = jnp.where(oh1, -jnp.inf, logits)
    i2 = jnp.argmax(masked, axis=1)[:, None]
    v2 = jnp.max(masked, axis=1)[:, None]
    oh2 = lane == i2
    e2 = jnp.exp(v2 - v1)
    denom = 1.0 + e2
    w_ref[...] = jnp.where(oh1, 1.0 / denom, jnp.where(oh2, e2 / denom, 0.0))

    outv = jnp.zeros_like(out_ref)
    h_cur = None
    h_prev = None
    acc = None

    for j in range(n_chunks + LAG):
        if j < n_chunks:
            e, q = j // 4, j % 4
            cp1(j, j % RING).wait()
            part = jnp.dot(xv[:, q * d_q:(q + 1) * d_q], buf1_ref[j % RING],
                           preferred_element_type=jnp.float32)
            h_cur = part if q == 0 else h_cur + part
            if q == 3:
                h_cur = h_cur + b1_ref[e]
        k = j - LAG
        if k >= 0:
            e2_, q2 = k // 4, k % 4
            cp2(k, k % RING).wait()
            rh = jnp.maximum(h_prev[:, q2 * h_q:(q2 + 1) * h_q], 0.0)
            term = jnp.dot(rh, buf2_ref[k % RING],
                           preferred_element_type=jnp.float32)
            acc = term if q2 == 0 else acc + term
            if q2 == 3:
                outv = outv + w_ref[:, e2_:e2_ + 1] * (acc + b2_ref[e2_])
        if j < n_chunks and j % 4 == 3:
            h_prev = h_cur
        if j + LOOK1 < n_chunks:
            cp1(j + LOOK1, (j + LOOK1) % RING).start()
        if j + 2 < n_chunks:
            cp2(j + 2, (j + 2) % RING).start()

    out_ref[...] = outv


def kernel(x, Wg, Wnoise, W1, b1, W2, b2):
    b, c, d = x.shape
    n_exp, _, d_hid = W1.shape
    t = b * c
    x2 = x.reshape(t, d)
    eps = jax.random.normal(jax.random.key(42), (b, c, n_exp),
                            dtype=x.dtype).reshape(t, n_exp)
    nc = 4 * n_exp
    w1r = W1.reshape(nc, d // 4, d_hid)
    w2r = W2.reshape(nc, d_hid // 4, d)
    out = pl.pallas_call(
        _moe_kernel,
        in_specs=[
            pl.BlockSpec((t, d), lambda: (0, 0)),
            pl.BlockSpec((d, n_exp), lambda: (0, 0)),
            pl.BlockSpec((d, n_exp), lambda: (0, 0)),
            pl.BlockSpec((t, n_exp), lambda: (0, 0)),
            pl.BlockSpec((n_exp, 1, d_hid), lambda: (0, 0, 0)),
            pl.BlockSpec((n_exp, 1, d), lambda: (0, 0, 0)),
            pl.BlockSpec(memory_space=pltpu.MemorySpace.HBM),
            pl.BlockSpec(memory_space=pltpu.MemorySpace.HBM),
        ],
        out_specs=pl.BlockSpec((t, d), lambda: (0, 0)),
        out_shape=jax.ShapeDtypeStruct((t, d), x.dtype),
        scratch_shapes=[
            pltpu.VMEM((t, n_exp), jnp.float32),
            pltpu.VMEM((RING, d // 4, d_hid), jnp.float32),
            pltpu.VMEM((RING, d_hid // 4, d), jnp.float32),
            pltpu.SemaphoreType.DMA((RING,)),
            pltpu.SemaphoreType.DMA((RING,)),
        ],
    )(x2, Wg.T, Wnoise.T, eps, b1[:, None, :], b2[:, None, :], w1r, w2r)
    return out.reshape(b, c, d)


# speedup vs baseline: 1.0316x; 1.0145x over previous
import jax
import jax.numpy as jnp
from jax.experimental import pallas as pl
from jax.experimental.pallas import tpu as pltpu

H_BLK = 1536


def _moe_kernel(x_ref, wg_ref, wn_ref, eps_ref, w1_ref, b1_ref, w2_ref, b2_ref,
                out_ref, acc_ref, w_ref):
    e = pl.program_id(0)
    c = pl.program_id(1)
    n_chunk = pl.num_programs(1)
    n_exp = wg_ref.shape[1]

    @pl.when((e == 0) & (c == 0))
    def _gating():
        xv = x_ref[...]
        g = jnp.dot(xv, wg_ref[...], preferred_element_type=jnp.float32)
        n = jnp.dot(xv, wn_ref[...], preferred_element_type=jnp.float32)
        logits = g + jax.nn.softplus(n) * eps_ref[...]
        lane = jax.lax.broadcasted_iota(jnp.int32, logits.shape, 1)
        i1 = jnp.argmax(logits, axis=1)[:, None]
        v1 = jnp.max(logits, axis=1)[:, None]
        oh1 = lane == i1
        masked = jnp.where(oh1, -jnp.inf, logits)
        i2 = jnp.argmax(masked, axis=1)[:, None]
        v2 = jnp.max(masked, axis=1)[:, None]
        oh2 = lane == i2
        e2 = jnp.exp(v2 - v1)
        denom = 1.0 + e2
        w_ref[...] = jnp.where(oh1, 1.0 / denom,
                               jnp.where(oh2, e2 / denom, 0.0))
        out_ref[...] = jnp.zeros_like(out_ref)

    @pl.when(c == 0)
    def _init_acc():
        acc_ref[...] = jnp.zeros_like(acc_ref)

    h = jnp.dot(x_ref[...], w1_ref[0], preferred_element_type=jnp.float32)
    h = jnp.maximum(h + b1_ref[0], 0.0)
    acc_ref[...] += jnp.dot(h, w2_ref[0], preferred_element_type=jnp.float32)

    @pl.when(c == n_chunk - 1)
    def _combine():
        lane = jax.lax.broadcasted_iota(jnp.int32, (out_ref.shape[0], n_exp), 1)
        we = jnp.sum(jnp.where(lane == e, w_ref[...], 0.0), axis=1,
                     keepdims=True)
        out_ref[...] += we * (acc_ref[...] + b2_ref[0])


def kernel(x, Wg, Wnoise, W1, b1, W2, b2):
    b, c, d = x.shape
    n_exp, _, d_hid = W1.shape
    t = b * c
    x2 = x.reshape(t, d)
    eps = jax.random.normal(jax.random.key(42), (b, c, n_exp),
                            dtype=x.dtype).reshape(t, n_exp)
    n_chunk = d_hid // H_BLK
    out = pl.pallas_call(
        _moe_kernel,
        grid=(n_exp, n_chunk),
        in_specs=[
            pl.BlockSpec((t, d), lambda e, c: (0, 0)),
            pl.BlockSpec((d, n_exp), lambda e, c: (0, 0)),
            pl.BlockSpec((d, n_exp), lambda e, c: (0, 0)),
            pl.BlockSpec((t, n_exp), lambda e, c: (0, 0)),
            pl.BlockSpec((1, d, H_BLK), lambda e, c: (e, 0, c)),
            pl.BlockSpec((1, 1, H_BLK), lambda e, c: (e, 0, c)),
            pl.BlockSpec((1, H_BLK, d), lambda e, c: (e, c, 0)),
            pl.BlockSpec((1, 1, d), lambda e, c: (e, 0, 0)),
        ],
        out_specs=pl.BlockSpec((t, d), lambda e, c: (0, 0)),
        out_shape=jax.ShapeDtypeStruct((t, d), x.dtype),
        scratch_shapes=[
            pltpu.VMEM((t, d), jnp.float32),
            pltpu.VMEM((t, n_exp), jnp.float32),
        ],
        compiler_params=pltpu.CompilerParams(
            dimension_semantics=("arbitrary", "arbitrary")),
    )(x2, Wg.T, Wnoise.T, eps, W1, b1[:, None, :], W2, b2[:, None, :])
    return out.reshape(b, c, d)


# speedup vs baseline: 1.0416x; 1.0096x over previous
import jax
import jax.numpy as jnp
from jax.experimental import pallas as pl
from jax.experimental.pallas import tpu as pltpu

H_BLK = 1536
RING = 4
LOOK = 2


def _moe_kernel(x_ref, wg_ref, wn_ref, eps_ref, w1_ref, b1_ref, b2_ref,
                w2_hbm, out_ref, acc_ref, w_ref, buf2_ref, sem2):
    e = pl.program_id(0)
    c = pl.program_id(1)
    n_chunk = pl.num_programs(1)
    n_exp = wg_ref.shape[1]
    n_chunks_total = w2_hbm.shape[0]
    i = e * n_chunk + c

    def cp2(idx, slot):
        return pltpu.make_async_copy(w2_hbm.at[pl.ds(idx, 1)],
                                     buf2_ref.at[pl.ds(slot, 1)],
                                     sem2.at[slot])

    @pl.when((e == 0) & (c == 0))
    def _gating():
        for k in range(LOOK + 1):
            cp2(k, k).start()
        xv = x_ref[...]
        g = jnp.dot(xv, wg_ref[...], preferred_element_type=jnp.float32)
        n = jnp.dot(xv, wn_ref[...], preferred_element_type=jnp.float32)
        logits = g + jax.nn.softplus(n) * eps_ref[...]
        lane = jax.lax.broadcasted_iota(jnp.int32, logits.shape, 1)
        i1 = jnp.argmax(logits, axis=1)[:, None]
        v1 = jnp.max(logits, axis=1)[:, None]
        oh1 = lane == i1
        masked = jnp.where(oh1, -jnp.inf, logits)
        i2 = jnp.argmax(masked, axis=1)[:, None]
        v2 = jnp.max(masked, axis=1)[:, None]
        oh2 = lane == i2
        e2 = jnp.exp(v2 - v1)
        denom = 1.0 + e2
        w_ref[...] = jnp.where(oh1, 1.0 / denom,
                               jnp.where(oh2, e2 / denom, 0.0))
        out_ref[...] = jnp.zeros_like(out_ref)

    @pl.when(c == 0)
    def _init_acc():
        acc_ref[...] = jnp.zeros_like(acc_ref)

    slot = jax.lax.rem(i, RING)
    cp2(i, slot).wait()

    h = jnp.dot(x_ref[...], w1_ref[0], preferred_element_type=jnp.float32)
    h = jnp.maximum(h + b1_ref[0], 0.0)
    for s in range(RING):
        @pl.when(slot == s)
        def _consume(s=s):
            acc_ref[...] += jnp.dot(h, buf2_ref[s],
                                    preferred_element_type=jnp.float32)

    @pl.when(i + LOOK + 1 < n_chunks_total)
    def _refill():
        cp2(i + LOOK + 1, jax.lax.rem(i + LOOK + 1, RING)).start()

    @pl.when(c == n_chunk - 1)
    def _combine():
        lane = jax.lax.broadcasted_iota(jnp.int32, (out_ref.shape[0], n_exp), 1)
        we = jnp.sum(jnp.where(lane == e, w_ref[...], 0.0), axis=1,
                     keepdims=True)
        out_ref[...] += we * (acc_ref[...] + b2_ref[0])


def kernel(x, Wg, Wnoise, W1, b1, W2, b2):
    b, c, d = x.shape
    n_exp, _, d_hid = W1.shape
    t = b * c
    x2 = x.reshape(t, d)
    eps = jax.random.normal(jax.random.key(42), (b, c, n_exp),
                            dtype=x.dtype).reshape(t, n_exp)
    n_chunk = d_hid // H_BLK
    w2r = W2.reshape(n_exp * n_chunk, H_BLK, d)
    out = pl.pallas_call(
        _moe_kernel,
        grid=(n_exp, n_chunk),
        in_specs=[
            pl.BlockSpec((t, d), lambda e, c: (0, 0)),
            pl.BlockSpec((d, n_exp), lambda e, c: (0, 0)),
            pl.BlockSpec((d, n_exp), lambda e, c: (0, 0)),
            pl.BlockSpec((t, n_exp), lambda e, c: (0, 0)),
            pl.BlockSpec((1, d, H_BLK), lambda e, c: (e, 0, c)),
            pl.BlockSpec((1, 1, H_BLK), lambda e, c: (e, 0, c)),
            pl.BlockSpec((1, 1, d), lambda e, c: (e, 0, 0)),
            pl.BlockSpec(memory_space=pltpu.MemorySpace.HBM),
        ],
        out_specs=pl.BlockSpec((t, d), lambda e, c: (0, 0)),
        out_shape=jax.ShapeDtypeStruct((t, d), x.dtype),
        scratch_shapes=[
            pltpu.VMEM((t, d), jnp.float32),
            pltpu.VMEM((t, n_exp), jnp.float32),
            pltpu.VMEM((RING, H_BLK, d), jnp.float32),
            pltpu.SemaphoreType.DMA((RING,)),
        ],
        compiler_params=pltpu.CompilerParams(
            dimension_semantics=("arbitrary", "arbitrary")),
    )(x2, Wg.T, Wnoise.T, eps, W1, b1[:, None, :], b2[:, None, :], w2r)
    return out.reshape(b, c, d)
